# Initial kernel scaffold; baseline (speedup 1.0000x reference)
import jax
import jax.numpy as jnp
from jax import lax
from jax.experimental import pallas as pl
from jax.experimental.pallas import tpu as pltpu
from jax.experimental.pallas import tpu_sc as plsc

USER_N = 20000
ITEM_N = 30000
NODES = USER_N + ITEM_N
EDGES = 800000
DIM = 64
HALF = DIM // 2
LAYERS = 3

NC = 2
NS = 16
CH = 80
K = 5
EROWS = EDGES // CH
ROWS_PER_SUB = EROWS // NS
STEPS = ROWS_PER_SUB // K
ACC_ROWS_PER_SUB = NODES // NS
ZROWS = 125


def _layer_body(lo_in, hi_in, rows_e, cols_e, w_e, lo_out, hi_out,
                acc, colv, rowv, wv, gbuf, zbuf, sem):
  c = lax.axis_index("c")
  s = lax.axis_index("s")

  zero16 = jnp.zeros((16,), jnp.float32)

  @pl.loop(0, ZROWS)
  def _(r):
    zbuf[r, 0:16] = zero16
    zbuf[r, 16:32] = zero16

  acc_base = s * ACC_ROWS_PER_SUB
  for i in range(ACC_ROWS_PER_SUB // ZROWS):
    pltpu.sync_copy(zbuf, acc.at[pl.ds(acc_base + i * ZROWS, ZROWS)])

  plsc.subcore_barrier()

  ebase = s * ROWS_PER_SUB

  @pl.loop(0, STEPS)
  def _(i):
    r0 = ebase + i * K
    pltpu.sync_copy(cols_e.at[pl.ds(r0, K)], colv)
    pltpu.sync_copy(rows_e.at[pl.ds(r0, K)], rowv)
    pltpu.sync_copy(w_e.at[pl.ds(r0, K)], wv)

    @pl.when(c == 0)
    def _():
      for k in range(K):
        pltpu.async_copy(lo_in.at[colv.at[k]], gbuf.at[k], sem)

    @pl.when(c == 1)
    def _():
      for k in range(K):
        pltpu.async_copy(hi_in.at[colv.at[k]], gbuf.at[k], sem)

    for k in range(K):
      pltpu.make_async_copy(lo_in.at[colv.at[k]], gbuf.at[k], sem).wait()

    for k in range(K):
      @pl.loop(0, CH)
      def _(j):
        wj = wv[k, j]
        gbuf[k, j, 0:16] = gbuf[k, j, 0:16] * wj
        gbuf[k, j, 16:32] = gbuf[k, j, 16:32] * wj

    for k in range(K):
      pltpu.sync_copy(gbuf.at[k], acc.at[rowv.at[k]], add=True)

  plsc.subcore_barrier()

  @pl.when(c == 0)
  def _():
    pltpu.sync_copy(acc.at[pl.ds(acc_base, ACC_ROWS_PER_SUB)],
                    lo_out.at[pl.ds(acc_base, ACC_ROWS_PER_SUB)])

  @pl.when(c == 1)
  def _():
    pltpu.sync_copy(acc.at[pl.ds(acc_base, ACC_ROWS_PER_SUB)],
                    hi_out.at[pl.ds(acc_base, ACC_ROWS_PER_SUB)])


_sc_layer = pl.kernel(
    _layer_body,
    out_type=(
        jax.ShapeDtypeStruct((NODES, HALF), jnp.float32),
        jax.ShapeDtypeStruct((NODES, HALF), jnp.float32),
    ),
    mesh=plsc.VectorSubcoreMesh(core_axis_name="c", subcore_axis_name="s"),
    scratch_types=[
        pltpu.VMEM_SHARED((NODES, HALF), jnp.float32),
        pltpu.VMEM((K, CH), jnp.int32),
        pltpu.VMEM((K, CH), jnp.int32),
        pltpu.VMEM((K, CH), jnp.float32),
        pltpu.VMEM((K, CH, HALF), jnp.float32),
        pltpu.VMEM((ZROWS, HALF), jnp.float32),
        pltpu.SemaphoreType.DMA,
    ],
)


def _mean_body(l1, l2, l3, h1, h2, h3, mlo, mhi):
  third = jnp.float32(1.0 / 3.0)
  mlo[...] = (l1[...] + l2[...] + l3[...]) * third
  mhi[...] = (h1[...] + h2[...] + h3[...]) * third


_BN = 500


def _mean3(l1, l2, l3, h1, h2, h3):
  packed = [x.reshape(NODES // 4, 128) for x in (l1, l2, l3, h1, h2, h3)]
  spec = pl.BlockSpec((_BN, 128), lambda i: (i, 0))
  mlo, mhi = pl.pallas_call(
      _mean_body,
      grid=(NODES // 4 // _BN,),
      in_specs=[spec] * 6,
      out_specs=[spec, spec],
      out_shape=[jax.ShapeDtypeStruct((NODES // 4, 128), jnp.float32)] * 2,
  )(*packed)
  return mlo.reshape(NODES, HALF), mhi.reshape(NODES, HALF)


@jax.jit
def kernel(user_emb, item_emb, edge_index, edge_weight):
  ego = jnp.concatenate([user_emb, item_emb], axis=0)
  lo = ego[:, :HALF]
  hi = ego[:, HALF:]
  rows_e = edge_index[0].reshape(EROWS, CH)
  cols_e = edge_index[1].reshape(EROWS, CH)
  w_e = edge_weight.reshape(EROWS, CH)

  outs = []
  for _ in range(LAYERS):
    lo, hi = _sc_layer(lo, hi, rows_e, cols_e, w_e)
    outs.append((lo, hi))

  mlo, mhi = _mean3(outs[0][0], outs[1][0], outs[2][0],
                    outs[0][1], outs[1][1], outs[2][1])
  all_emb = jnp.concatenate([mlo, mhi], axis=1)
  return all_emb[:USER_N], all_emb[USER_N:]


# speedup vs baseline: 3.1736x; 3.1736x over previous
import jax
import jax.numpy as jnp
from jax import lax
from jax.experimental import pallas as pl
from jax.experimental.pallas import tpu as pltpu
from jax.experimental.pallas import tpu_sc as plsc

USER_N = 20000
ITEM_N = 30000
NODES = USER_N + ITEM_N
EDGES = 800000
DIM = 64
HALF = DIM // 2
LAYERS = 3

NC = 2
NS = 16
CH = 80
K = 5
EROWS = EDGES // CH
ROWS_PER_SUB = EROWS // NS
STEPS = ROWS_PER_SUB // K
NPAD = 50048
ACC_ROWS_PER_SUB = NPAD // NS
ZROWS = 136


def _layer_body(lo_in, hi_in, rows_e, cols_e, w_e, lo_out, hi_out,
                acc, colv, rowv, wv, gbuf, zbuf, sem):
  c = lax.axis_index("c")
  s = lax.axis_index("s")

  zero16 = jnp.zeros((16,), jnp.float32)

  @pl.loop(0, ZROWS)
  def _(r):
    zbuf[r, 0:16] = zero16
    zbuf[r, 16:32] = zero16

  acc_base = s * ACC_ROWS_PER_SUB
  for i in range(ACC_ROWS_PER_SUB // ZROWS):
    pltpu.sync_copy(zbuf, acc.at[pl.ds(acc_base + i * ZROWS, ZROWS)])

  plsc.subcore_barrier()

  ebase = s * (EDGES // NS)

  @pl.loop(0, STEPS)
  def _(i):
    e0 = ebase + i * (K * CH)
    for k in range(K):
      pltpu.sync_copy(cols_e.at[pl.ds(e0 + k * CH, CH)], colv.at[k])
      pltpu.sync_copy(rows_e.at[pl.ds(e0 + k * CH, CH)], rowv.at[k])
      pltpu.sync_copy(w_e.at[pl.ds(e0 + k * CH, CH)], wv.at[k])

    @pl.when(c == 0)
    def _():
      for k in range(K):
        pltpu.async_copy(lo_in.at[colv.at[k]], gbuf.at[k], sem)

    @pl.when(c == 1)
    def _():
      for k in range(K):
        pltpu.async_copy(hi_in.at[colv.at[k]], gbuf.at[k], sem)

    for k in range(K):
      pltpu.make_async_copy(lo_in.at[colv.at[k]], gbuf.at[k], sem).wait()

    for k in range(K):
      @pl.loop(0, CH // 16)
      def _(g):
        j0 = g * 16
        wrow = wv[k, pl.ds(j0, 16)]
        for jj in range(16):
          j = j0 + jj
          wj = wrow[jj]
          gbuf[k, j, 0:16] = gbuf[k, j, 0:16] * wj
          gbuf[k, j, 16:32] = gbuf[k, j, 16:32] * wj

    for k in range(K):
      pltpu.sync_copy(gbuf.at[k], acc.at[rowv.at[k]], add=True)

  plsc.subcore_barrier()

  @pl.when(c == 0)
  def _():
    pltpu.sync_copy(acc.at[pl.ds(acc_base, ACC_ROWS_PER_SUB)],
                    lo_out.at[pl.ds(acc_base, ACC_ROWS_PER_SUB)])

  @pl.when(c == 1)
  def _():
    pltpu.sync_copy(acc.at[pl.ds(acc_base, ACC_ROWS_PER_SUB)],
                    hi_out.at[pl.ds(acc_base, ACC_ROWS_PER_SUB)])


_sc_layer = pl.kernel(
    _layer_body,
    out_type=(
        jax.ShapeDtypeStruct((NPAD, HALF), jnp.float32),
        jax.ShapeDtypeStruct((NPAD, HALF), jnp.float32),
    ),
    mesh=plsc.VectorSubcoreMesh(core_axis_name="c", subcore_axis_name="s"),
    compiler_params=pltpu.CompilerParams(use_tc_tiling_on_sc=False),
    scratch_types=[
        pltpu.VMEM_SHARED((NPAD, HALF), jnp.float32),
        pltpu.VMEM((K, CH), jnp.int32),
        pltpu.VMEM((K, CH), jnp.int32),
        pltpu.VMEM((K, CH), jnp.float32),
        pltpu.VMEM((K, CH, HALF), jnp.float32),
        pltpu.VMEM((ZROWS, HALF), jnp.float32),
        pltpu.SemaphoreType.DMA,
    ],
)


def _mean_body(l1, l2, l3, h1, h2, h3, mlo, mhi):
  third = jnp.float32(1.0 / 3.0)
  mlo[...] = (l1[...] + l2[...] + l3[...]) * third
  mhi[...] = (h1[...] + h2[...] + h3[...]) * third


_MR = NPAD * HALF // 128
_MBN = 136


def _mean3(l1, l2, l3, h1, h2, h3):
  packed = [x.reshape(_MR, 128) for x in (l1, l2, l3, h1, h2, h3)]
  spec = pl.BlockSpec((_MBN, 128), lambda i: (i, 0))
  mlo, mhi = pl.pallas_call(
      _mean_body,
      grid=(_MR // _MBN,),
      in_specs=[spec] * 6,
      out_specs=[spec, spec],
      out_shape=[jax.ShapeDtypeStruct((_MR, 128), jnp.float32)] * 2,
  )(*packed)
  return (mlo.reshape(NPAD, HALF)[:NODES], mhi.reshape(NPAD, HALF)[:NODES])


@jax.jit
def kernel(user_emb, item_emb, edge_index, edge_weight):
  ego = jnp.concatenate([user_emb, item_emb], axis=0)
  ego = jnp.pad(ego, ((0, NPAD - NODES), (0, 0)))
  lo = ego[:, :HALF]
  hi = ego[:, HALF:]
  rows_e = edge_index[0]
  cols_e = edge_index[1]
  w_e = edge_weight

  outs = []
  for _ in range(LAYERS):
    lo, hi = _sc_layer(lo, hi, rows_e, cols_e, w_e)
    outs.append((lo, hi))

  mlo, mhi = _mean3(outs[0][0], outs[1][0], outs[2][0],
                    outs[0][1], outs[1][1], outs[2][1])
  all_emb = jnp.concatenate([mlo, mhi], axis=1)
  return all_emb[:USER_N], all_emb[USER_N:]


# speedup vs baseline: 6.3964x; 2.0155x over previous
import jax
import jax.numpy as jnp
from jax import lax
from jax.experimental import pallas as pl
from jax.experimental.pallas import tpu as pltpu
from jax.experimental.pallas import tpu_sc as plsc

USER_N = 20000
ITEM_N = 30000
NODES = USER_N + ITEM_N
EDGES = 800000
DIM = 64
HALF = DIM // 2
LAYERS = 3

NC = 2
NS = 16
CH = 80
K = 4
MACRO = K * CH
STEPS = 160
EPS = MACRO * STEPS
EPAD = EPS * NS
MACROS = EPAD // MACRO
NPAD = 50048
ACC_ROWS_PER_SUB = NPAD // NS
ZROWS = 136


def _layer_body(lo_in, hi_in, edges_p, lo_out, hi_out,
                acc, eb0, eb1, gb0, gb1, zbuf, sem_e, sem_g, sem_s):
  c = lax.axis_index("c")
  s = lax.axis_index("s")
  ebufs = (eb0, eb1)
  gbufs = (gb0, gb1)
  src = (lo_in, hi_in)

  zero16 = jnp.zeros((16,), jnp.float32)

  @pl.loop(0, ZROWS)
  def _(r):
    zbuf[r, 0:16] = zero16
    zbuf[r, 16:32] = zero16

  acc_base = s * ACC_ROWS_PER_SUB
  for i in range(ACC_ROWS_PER_SUB // ZROWS):
    pltpu.sync_copy(zbuf, acc.at[pl.ds(acc_base + i * ZROWS, ZROWS)])

  plsc.subcore_barrier()

  m0 = s * STEPS

  def fire_edges(buf_ix, step):
    pltpu.async_copy(edges_p.at[m0 + step], ebufs[buf_ix], sem_e)

  def wait_edges(buf_ix, step):
    pltpu.make_async_copy(edges_p.at[m0 + step], ebufs[buf_ix], sem_e).wait()

  def fire_gathers(buf_ix):
    eb, gb = ebufs[buf_ix], gbufs[buf_ix]
    for ci in range(NC):
      @pl.when(c == ci)
      def _():
        for k in range(K):
          pltpu.async_copy(src[ci].at[eb.at[K + k]], gb.at[k], sem_g)

  def wait_gathers(buf_ix):
    eb, gb = ebufs[buf_ix], gbufs[buf_ix]
    for k in range(K):
      pltpu.make_async_copy(lo_in.at[eb.at[K + k]], gb.at[k], sem_g).wait()

  def scale(buf_ix):
    eb, gb = ebufs[buf_ix], gbufs[buf_ix]
    for k in range(K):
      @pl.loop(0, CH // 16)
      def _(g):
        j0 = g * 16
        wrow = plsc.bitcast(eb[2 * K + k, pl.ds(j0, 16)], jnp.float32)
        for jj in range(16):
          j = j0 + jj
          wj = wrow[jj]
          gb[k, j, 0:16] = gb[k, j, 0:16] * wj
          gb[k, j, 16:32] = gb[k, j, 16:32] * wj

  def fire_scatters(buf_ix):
    eb, gb = ebufs[buf_ix], gbufs[buf_ix]
    for k in range(K):
      pltpu.async_copy(gb.at[k], acc.at[eb.at[k]], sem_s, add=True)

  def wait_scatters(buf_ix):
    eb, gb = ebufs[buf_ix], gbufs[buf_ix]
    for k in range(K):
      pltpu.make_async_copy(gb.at[k], acc.at[eb.at[k]], sem_s).wait()

  fire_edges(0, 0)
  wait_edges(0, 0)
  fire_gathers(0)
  fire_edges(1, 1)

  @pl.loop(0, STEPS)
  def _(i):
    cur = lax.rem(i, 2)
    for b in range(2):
      @pl.when(cur == b)
      def _():
        nxt = 1 - b
        wait_gathers(b)

        @pl.when(i + 1 < STEPS)
        def _():
          wait_edges(nxt, i + 1)

        scale(b)

        @pl.when(i >= 1)
        def _():
          wait_scatters(nxt)

        @pl.when(i + 1 < STEPS)
        def _():
          fire_gathers(nxt)

        @pl.when(i + 2 < STEPS)
        def _():
          fire_edges(b, i + 2)

        fire_scatters(b)

  wait_scatters((STEPS - 1) % 2)

  plsc.subcore_barrier()

  @pl.when(c == 0)
  def _():
    pltpu.sync_copy(acc.at[pl.ds(acc_base, ACC_ROWS_PER_SUB)],
                    lo_out.at[pl.ds(acc_base, ACC_ROWS_PER_SUB)])

  @pl.when(c == 1)
  def _():
    pltpu.sync_copy(acc.at[pl.ds(acc_base, ACC_ROWS_PER_SUB)],
                    hi_out.at[pl.ds(acc_base, ACC_ROWS_PER_SUB)])


_sc_layer = pl.kernel(
    _layer_body,
    out_type=(
        jax.ShapeDtypeStruct((NPAD, HALF), jnp.float32),
        jax.ShapeDtypeStruct((NPAD, HALF), jnp.float32),
    ),
    mesh=plsc.VectorSubcoreMesh(core_axis_name="c", subcore_axis_name="s"),
    compiler_params=pltpu.CompilerParams(use_tc_tiling_on_sc=False,
                                         needs_layout_passes=False),
    scratch_types=[
        pltpu.VMEM_SHARED((NPAD, HALF), jnp.float32),
        pltpu.VMEM((3 * K, CH), jnp.int32),
        pltpu.VMEM((3 * K, CH), jnp.int32),
        pltpu.VMEM((K, CH, HALF), jnp.float32),
        pltpu.VMEM((K, CH, HALF), jnp.float32),
        pltpu.VMEM((ZROWS, HALF), jnp.float32),
        pltpu.SemaphoreType.DMA,
        pltpu.SemaphoreType.DMA,
        pltpu.SemaphoreType.DMA,
    ],
)


def _mean_body(l1, l2, l3, h1, h2, h3, mlo, mhi):
  third = jnp.float32(1.0 / 3.0)
  mlo[...] = (l1[...] + l2[...] + l3[...]) * third
  mhi[...] = (h1[...] + h2[...] + h3[...]) * third


_MR = NPAD * HALF // 128
_MBN = 136


def _mean3(l1, l2, l3, h1, h2, h3):
  packed = [x.reshape(_MR, 128) for x in (l1, l2, l3, h1, h2, h3)]
  spec = pl.BlockSpec((_MBN, 128), lambda i: (i, 0))
  mlo, mhi = pl.pallas_call(
      _mean_body,
      grid=(_MR // _MBN,),
      in_specs=[spec] * 6,
      out_specs=[spec, spec],
      out_shape=[jax.ShapeDtypeStruct((_MR, 128), jnp.float32)] * 2,
  )(*packed)
  return (mlo.reshape(NPAD, HALF)[:NODES], mhi.reshape(NPAD, HALF)[:NODES])


def _pack_edges(edge_index, edge_weight):
  pad = EPAD - EDGES
  rows = jnp.pad(edge_index[0], (0, pad)).reshape(MACROS, K, CH)
  cols = jnp.pad(edge_index[1], (0, pad)).reshape(MACROS, K, CH)
  w32 = lax.bitcast_convert_type(jnp.pad(edge_weight, (0, pad)), jnp.int32)
  return jnp.concatenate([rows, cols, w32.reshape(MACROS, K, CH)], axis=1)


@jax.jit
def kernel(user_emb, item_emb, edge_index, edge_weight):
  ego = jnp.concatenate([user_emb, item_emb], axis=0)
  ego = jnp.pad(ego, ((0, NPAD - NODES), (0, 0)))
  lo = ego[:, :HALF]
  hi = ego[:, HALF:]
  edges_p = _pack_edges(edge_index, edge_weight)

  outs = []
  for _ in range(LAYERS):
    lo, hi = _sc_layer(lo, hi, edges_p)
    outs.append((lo, hi))

  mlo, mhi = _mean3(outs[0][0], outs[1][0], outs[2][0],
                    outs[0][1], outs[1][1], outs[2][1])
  all_emb = jnp.concatenate([mlo, mhi], axis=1)
  return all_emb[:USER_N], all_emb[USER_N:]


# speedup vs baseline: 8.8383x; 1.3818x over previous
import jax
import jax.numpy as jnp
from jax import lax
from jax.experimental import pallas as pl
from jax.experimental.pallas import tpu as pltpu
from jax.experimental.pallas import tpu_sc as plsc

USER_N = 20000
ITEM_N = 30000
NODES = USER_N + ITEM_N
EDGES = 800000
DIM = 64
HALF = DIM // 2
LAYERS = 3

NC = 2
NS = 16
CH = 128
RING = 5
STEPS = 391
EPAD = NS * STEPS * CH
MACROS = EPAD // CH
NPAD = 50048
ACC_ROWS_PER_SUB = NPAD // NS
ZROWS = 136


def _layer_body(lo_in, hi_in, edges_p, lo_out, hi_out, *scratch):
  acc = scratch[0]
  ebufs = scratch[1:1 + RING]
  gbufs = scratch[1 + RING:1 + 2 * RING]
  zbuf = scratch[1 + 2 * RING]
  sem_e, sem_g, sem_s = scratch[2 + 2 * RING:]
  c = lax.axis_index("c")
  s = lax.axis_index("s")
  src = (lo_in, hi_in)

  zero16 = jnp.zeros((16,), jnp.float32)

  @pl.loop(0, ZROWS)
  def _(r):
    zbuf[r, 0:16] = zero16
    zbuf[r, 16:32] = zero16

  acc_base = s * ACC_ROWS_PER_SUB
  for i in range(ACC_ROWS_PER_SUB // ZROWS):
    pltpu.sync_copy(zbuf, acc.at[pl.ds(acc_base + i * ZROWS, ZROWS)])

  plsc.subcore_barrier()

  m0 = s * STEPS

  def fire_edges(q, step):
    pltpu.async_copy(edges_p.at[m0 + step], ebufs[q], sem_e)

  def wait_edges(q, step):
    pltpu.make_async_copy(edges_p.at[m0 + step], ebufs[q], sem_e).wait()

  def fire_gathers(q):
    for ci in range(NC):
      @pl.when(c == ci)
      def _():
        pltpu.async_copy(src[ci].at[ebufs[q].at[1]], gbufs[q], sem_g)

  def wait_gathers(q):
    pltpu.make_async_copy(lo_in.at[ebufs[q].at[1]], gbufs[q], sem_g).wait()

  def scale(q):
    eb, gb = ebufs[q], gbufs[q]

    @pl.loop(0, CH // 16)
    def _(g):
      j0 = g * 16
      wrow = plsc.bitcast(eb[2, pl.ds(j0, 16)], jnp.float32)
      for jj in range(16):
        j = j0 + jj
        wj = wrow[jj]
        gb[j, 0:16] = gb[j, 0:16] * wj
        gb[j, 16:32] = gb[j, 16:32] * wj

  def fire_scatters(q):
    pltpu.async_copy(gbufs[q], acc.at[ebufs[q].at[0]], sem_s, add=True)

  def wait_scatters(q):
    pltpu.make_async_copy(gbufs[q], acc.at[ebufs[q].at[0]], sem_s).wait()

  fire_edges(0, 0)
  fire_edges(1, 1)
  wait_edges(0, 0)
  fire_gathers(0)

  @pl.loop(0, STEPS)
  def _(i):
    cur = lax.rem(i, RING)
    for b in range(RING):
      @pl.when(cur == b)
      def _():
        nx1 = (b + 1) % RING
        nx2 = (b + 2) % RING
        wait_gathers(b)

        @pl.when(i + 1 < STEPS)
        def _():
          wait_edges(nx1, i + 1)

        @pl.when(i >= RING - 2)
        def _():
          wait_scatters(nx2)

        @pl.when(i + 1 < STEPS)
        def _():
          fire_gathers(nx1)

        scale(b)

        @pl.when(i + 2 < STEPS)
        def _():
          fire_edges(nx2, i + 2)

        fire_scatters(b)

  for j in range(RING - 2, 0, -1):
    wait_scatters((STEPS - j) % RING)

  plsc.subcore_barrier()

  @pl.when(c == 0)
  def _():
    pltpu.sync_copy(acc.at[pl.ds(acc_base, ACC_ROWS_PER_SUB)],
                    lo_out.at[pl.ds(acc_base, ACC_ROWS_PER_SUB)])

  @pl.when(c == 1)
  def _():
    pltpu.sync_copy(acc.at[pl.ds(acc_base, ACC_ROWS_PER_SUB)],
                    hi_out.at[pl.ds(acc_base, ACC_ROWS_PER_SUB)])


_sc_layer = pl.kernel(
    _layer_body,
    out_type=(
        jax.ShapeDtypeStruct((NPAD, HALF), jnp.float32),
        jax.ShapeDtypeStruct((NPAD, HALF), jnp.float32),
    ),
    mesh=plsc.VectorSubcoreMesh(core_axis_name="c", subcore_axis_name="s"),
    compiler_params=pltpu.CompilerParams(use_tc_tiling_on_sc=False,
                                         needs_layout_passes=False),
    scratch_types=(
        [pltpu.VMEM_SHARED((NPAD, HALF), jnp.float32)]
        + [pltpu.VMEM((3, CH), jnp.int32)] * RING
        + [pltpu.VMEM((CH, HALF), jnp.float32)] * RING
        + [pltpu.VMEM((ZROWS, HALF), jnp.float32)]
        + [pltpu.SemaphoreType.DMA] * 3
    ),
)


def _mean_body(l1, l2, l3, h1, h2, h3, mlo, mhi):
  third = jnp.float32(1.0 / 3.0)
  mlo[...] = (l1[...] + l2[...] + l3[...]) * third
  mhi[...] = (h1[...] + h2[...] + h3[...]) * third


_MR = NPAD * HALF // 128
_MBN = 136


def _mean3(l1, l2, l3, h1, h2, h3):
  packed = [x.reshape(_MR, 128) for x in (l1, l2, l3, h1, h2, h3)]
  spec = pl.BlockSpec((_MBN, 128), lambda i: (i, 0))
  mlo, mhi = pl.pallas_call(
      _mean_body,
      grid=(_MR // _MBN,),
      in_specs=[spec] * 6,
      out_specs=[spec, spec],
      out_shape=[jax.ShapeDtypeStruct((_MR, 128), jnp.float32)] * 2,
  )(*packed)
  return (mlo.reshape(NPAD, HALF)[:NODES], mhi.reshape(NPAD, HALF)[:NODES])


def _pack_edges(edge_index, edge_weight):
  pad = EPAD - EDGES
  rows = jnp.pad(edge_index[0], (0, pad)).reshape(MACROS, 1, CH)
  cols = jnp.pad(edge_index[1], (0, pad)).reshape(MACROS, 1, CH)
  w32 = lax.bitcast_convert_type(jnp.pad(edge_weight, (0, pad)), jnp.int32)
  return jnp.concatenate([rows, cols, w32.reshape(MACROS, 1, CH)], axis=1)


@jax.jit
def kernel(user_emb, item_emb, edge_index, edge_weight):
  ego = jnp.concatenate([user_emb, item_emb], axis=0)
  ego = jnp.pad(ego, ((0, NPAD - NODES), (0, 0)))
  lo = ego[:, :HALF]
  hi = ego[:, HALF:]
  edges_p = _pack_edges(edge_index, edge_weight)

  outs = []
  for _ in range(LAYERS):
    lo, hi = _sc_layer(lo, hi, edges_p)
    outs.append((lo, hi))

  mlo, mhi = _mean3(outs[0][0], outs[1][0], outs[2][0],
                    outs[0][1], outs[1][1], outs[2][1])
  all_emb = jnp.concatenate([mlo, mhi], axis=1)
  return all_emb[:USER_N], all_emb[USER_N:]
